# baseline (speedup 1.0000x reference)
import numpy as np
import jax
import jax.numpy as jnp
from jax.experimental import pallas as pl
from jax.experimental.pallas import tpu as pltpu


def _positional_table(T, E):
    pos = np.arange(T, dtype=np.float32)[:, None]
    i = np.arange(E, dtype=np.float32)[None, :]
    angles = pos / np.power(10000.0, 2.0 * i / E)
    table = np.array(angles, dtype=np.float32)
    table[:, 0::2] = np.sin(table[:, 0::2])
    table[:, 1::2] = np.cos(table[:, 1::2])
    return table


def _add_kernel(x_ref, t_ref, o_ref):
    o_ref[...] = x_ref[...] + t_ref[...]


def kernel(x):
    B, T, E = x.shape
    table = jnp.asarray(_positional_table(T, E))
    BS = 64
    out = pl.pallas_call(
        _add_kernel,
        grid=(B // BS,),
        in_specs=[
            pl.BlockSpec((BS, T, E), lambda i: (i, 0, 0)),
            pl.BlockSpec((T, E), lambda i: (0, 0)),
        ],
        out_specs=pl.BlockSpec((BS, T, E), lambda i: (i, 0, 0)),
        out_shape=jax.ShapeDtypeStruct((B, T, E), x.dtype),
        compiler_params=pltpu.CompilerParams(
            dimension_semantics=("parallel",),
        ),
    )(x, table)
    return out


# speedup vs baseline: 1.6754x; 1.6754x over previous
import numpy as np
import jax
import jax.numpy as jnp
from jax.experimental import pallas as pl
from jax.experimental.pallas import tpu as pltpu


def _positional_table(T, E):
    pos = np.arange(T, dtype=np.float32)[:, None]
    i = np.arange(E, dtype=np.float32)[None, :]
    angles = pos / np.power(10000.0, 2.0 * i / E)
    table = np.array(angles, dtype=np.float32)
    table[:, 0::2] = np.sin(table[:, 0::2])
    table[:, 1::2] = np.cos(table[:, 1::2])
    return table


def _add_kernel(x_ref, t_ref, o_ref):
    o_ref[...] = x_ref[...] + t_ref[...]


def kernel(x):
    B, T, E = x.shape
    table = jnp.asarray(_positional_table(T, E).reshape(1, T * E))
    x2 = x.reshape(B, T * E)
    BS = 256
    out = pl.pallas_call(
        _add_kernel,
        grid=(B // BS,),
        in_specs=[
            pl.BlockSpec((BS, T * E), lambda i: (i, 0)),
            pl.BlockSpec((1, T * E), lambda i: (0, 0)),
        ],
        out_specs=pl.BlockSpec((BS, T * E), lambda i: (i, 0)),
        out_shape=jax.ShapeDtypeStruct((B, T * E), x.dtype),
        compiler_params=pltpu.CompilerParams(
            dimension_semantics=("parallel",),
        ),
    )(x2, table)
    return out.reshape(B, T, E)
